# baseline (speedup 1.0000x reference)
import jax
import jax.numpy as jnp
from jax.experimental import pallas as pl
from jax.experimental.pallas import tpu as pltpu

_TILE_M = 768
_TILE_N = 8192


def _cosine_logits_kernel(h_ref, w_ref, o_ref):
    h = h_ref[...]
    w = w_ref[...]
    hn = h * jax.lax.rsqrt(jnp.maximum(jnp.sum(h * h, axis=-1, keepdims=True), 1e-24))
    wn = w * jax.lax.rsqrt(jnp.maximum(jnp.sum(w * w, axis=-1, keepdims=True), 1e-24))
    o_ref[...] = jax.lax.dot_general(
        hn.astype(jnp.bfloat16), wn.astype(jnp.bfloat16),
        dimension_numbers=(((1,), (1,)), ((), ())),
        preferred_element_type=jnp.float32,
    )


def kernel(hidden, codebook):
    b, n, d = hidden.shape
    k, _ = codebook.shape
    m = b * n
    h2 = hidden.reshape(m, d)

    grid = (m // _TILE_M, k // _TILE_N)
    out = pl.pallas_call(
        _cosine_logits_kernel,
        grid=grid,
        in_specs=[
            pl.BlockSpec((_TILE_M, d), lambda i, j: (i, 0)),
            pl.BlockSpec((_TILE_N, d), lambda i, j: (j, 0)),
        ],
        out_specs=pl.BlockSpec((_TILE_M, _TILE_N), lambda i, j: (i, j)),
        out_shape=jax.ShapeDtypeStruct((m, k), jnp.float32),
        compiler_params=pltpu.CompilerParams(
            dimension_semantics=("parallel", "parallel"),
        ),
    )(h2, codebook)
    return out.reshape(b, n, k)


# speedup vs baseline: 1.0350x; 1.0350x over previous
import jax
import jax.numpy as jnp
from jax.experimental import pallas as pl
from jax.experimental.pallas import tpu as pltpu

_TILE_M = 256
_TILE_N = 8192


def _cosine_logits_kernel(h_ref, w_ref, o_ref):
    h = h_ref[...]
    w = w_ref[...]
    hn = h * jax.lax.rsqrt(jnp.maximum(jnp.sum(h * h, axis=-1, keepdims=True), 1e-24))
    wn = w * jax.lax.rsqrt(jnp.maximum(jnp.sum(w * w, axis=-1, keepdims=True), 1e-24))
    o_ref[...] = jax.lax.dot_general(
        hn.astype(jnp.bfloat16), wn.astype(jnp.bfloat16),
        dimension_numbers=(((1,), (1,)), ((), ())),
        preferred_element_type=jnp.float32,
    )


def kernel(hidden, codebook):
    b, n, d = hidden.shape
    k, _ = codebook.shape
    m = b * n
    h2 = hidden.reshape(m, d)

    grid = (m // _TILE_M, k // _TILE_N)
    out = pl.pallas_call(
        _cosine_logits_kernel,
        grid=grid,
        in_specs=[
            pl.BlockSpec((_TILE_M, d), lambda i, j: (i, 0)),
            pl.BlockSpec((_TILE_N, d), lambda i, j: (j, 0)),
        ],
        out_specs=pl.BlockSpec((_TILE_M, _TILE_N), lambda i, j: (i, j)),
        out_shape=jax.ShapeDtypeStruct((m, k), jnp.float32),
        compiler_params=pltpu.CompilerParams(
            dimension_semantics=("parallel", "parallel"),
        ),
    )(h2, codebook)
    return out.reshape(b, n, k)
